# Initial kernel scaffold; baseline (speedup 1.0000x reference)
import jax
import jax.numpy as jnp
from jax.experimental import pallas as pl


def kernel(box_cls_feat_con, crop_feat_con, batch_size, ious):
    raise NotImplementedError("write your pallas kernel here")


# speedup vs baseline: 11.3398x; 11.3398x over previous
import functools

import jax
import jax.numpy as jnp
from jax.experimental import pallas as pl
from jax.experimental.pallas import tpu as pltpu

_TEMP = 0.2
_THRES = 0.4
_NPI = 256
_D = 512
_NB = 32
_NLVL = 3


def _stage1_body(iou_ref, crop_ref, box_ref, out_ref):
    x = box_ref[...]
    z = crop_ref[0]
    nb2 = jnp.sum(x * x, axis=1, keepdims=True)
    inv_nb = jax.lax.rsqrt(jnp.maximum(nb2, 1e-24))
    nz2 = jnp.sum(z * z, axis=1, keepdims=True)
    inv_nz = jax.lax.rsqrt(jnp.maximum(nz2, 1e-24))
    zh = z * inv_nz
    dots = jax.lax.dot_general(
        x, zh, (((1,), (1,)), ((), ())),
        preferred_element_type=jnp.float32)
    cos = dots * inv_nb
    mask = (iou_ref[0] >= _THRES).astype(jnp.float32)
    sp = jnp.sum(cos * mask, axis=0, keepdims=True)
    sa = jnp.sum(cos, axis=0, keepdims=True)
    cp = jnp.sum(mask)
    lane = jax.lax.broadcasted_iota(jnp.int32, (1, 128), 1)
    row = jnp.zeros((1, 128), jnp.float32)
    for k in range(_NLVL):
        row = jnp.where(lane == 16 * k, sp[0, k], row)
        row = jnp.where(lane == 16 * (k + 3), sa[0, k], row)
    row = jnp.where(lane == 16 * 6, cp, row)
    out_ref[0] = row


def _stage2_body(stats_ref, binv_ref, out_ref):
    s = stats_ref[:, 0, :]
    cp = s[:, 96:97]
    cn = _NPI - cp
    lvl_tot = None
    for k in range(_NLVL):
        sp = s[:, 16 * k:16 * k + 1]
        sa = s[:, 16 * (k + 3):16 * (k + 3) + 1]
        sn = sa - sp
        sim_pos = -(sp / cp)
        sim_neg = -(sn / cn)
        pos = jnp.exp(sim_pos / _TEMP)
        neg = jnp.exp(sim_neg / _TEMP)
        lb = -jnp.log(pos / (pos + neg))
        lvl = jnp.sum(lb, axis=0, keepdims=True)
        lvl_tot = lvl if lvl_tot is None else jnp.minimum(lvl_tot, lvl)
    out_ref[...] = lvl_tot * binv_ref[0, 0]


def _stage1_tc(box, crop, iou3):
    return pl.pallas_call(
        _stage1_body,
        grid=(_NB,),
        in_specs=[
            pl.BlockSpec((1, _NPI, 1), lambda b: (b, 0, 0)),
            pl.BlockSpec((1, _NLVL, _D), lambda b: (b, 0, 0)),
            pl.BlockSpec((_NPI, _D), lambda b: (b, 0)),
        ],
        out_specs=pl.BlockSpec((1, 1, 128), lambda b: (b, 0, 0)),
        out_shape=jax.ShapeDtypeStruct((_NB, 1, 128), jnp.float32),
    )(iou3, crop, box)


def _stage2(stats, binv):
    return pl.pallas_call(
        _stage2_body,
        in_specs=[
            pl.BlockSpec((_NB, 1, 128), lambda: (0, 0, 0)),
            pl.BlockSpec(memory_space=pltpu.SMEM),
        ],
        out_specs=pl.BlockSpec((1, 1), lambda: (0, 0)),
        out_shape=jax.ShapeDtypeStruct((1, 1), jnp.float32),
    )(stats, binv)


def kernel(box_cls_feat_con, crop_feat_con, batch_size, ious):
    iou3 = ious.reshape(_NB, _NPI, 1)
    crop_feat_con = jnp.transpose(crop_feat_con, (1, 0, 2))
    binv = (1.0 / jnp.asarray(batch_size, jnp.float32)).reshape(1, 1)
    stats = _stage1_tc(box_cls_feat_con, crop_feat_con, iou3)
    loss = _stage2(stats, binv)
    return loss[0, 0]
